# Initial kernel scaffold; baseline (speedup 1.0000x reference)
import jax
import jax.numpy as jnp
from jax.experimental import pallas as pl


def kernel(center, context, emb_in, W_out):
    raise NotImplementedError("write your pallas kernel here")


# speedup vs baseline: 3.9542x; 3.9542x over previous
import functools

import jax
import jax.numpy as jnp
from jax import lax
from jax.experimental import pallas as pl
from jax.experimental.pallas import tpu as pltpu
from jax.experimental.pallas import tpu_sc as plsc

V = 1000
D = 128
B = 1024
L = 20
VP = 1024
LP = 32
N = B * L

_INFO = plsc.get_sparse_core_info()
_NC = _INFO.num_cores
_NS = _INFO.num_subcores
LN = _INFO.num_lanes
NW = _NC * _NS
BW = B // NW


def _ht_body(emb_ref, w_ref, ht_ref):
    a = lax.dot_general(w_ref[...], emb_ref[...], (((1,), (1,)), ((), ())),
                        preferred_element_type=jnp.float32)
    row_c = lax.broadcasted_iota(jnp.int32, (VP, VP), 0)
    a_msk = jnp.where(row_c < V, a, -1e30)
    m = jnp.max(a_msk, axis=0, keepdims=True)
    lz = m + jnp.log(jnp.sum(jnp.exp(a_msk - m), axis=0, keepdims=True))
    col_x = lax.broadcasted_iota(jnp.int32, (VP, VP), 1)
    ht_ref[...] = jnp.where(col_x < V, lz - a, 0.0)


_ht_call = pl.pallas_call(
    _ht_body,
    out_shape=jax.ShapeDtypeStruct((VP, VP), jnp.float32),
)


_mesh = plsc.VectorSubcoreMesh(core_axis_name="c", subcore_axis_name="s")


@functools.partial(
    pl.kernel,
    mesh=_mesh,
    compiler_params=pltpu.CompilerParams(use_tc_tiling_on_sc=False,
                                         needs_layout_passes=False),
    out_type=jax.ShapeDtypeStruct((NW, LN), jnp.float32),
    scratch_types=[
        pltpu.VMEM((BW,), jnp.int32),
        pltpu.VMEM((BW, LP), jnp.int32),
        pltpu.VMEM((BW, VP), jnp.float32),
        pltpu.VMEM((LN,), jnp.float32),
        pltpu.SemaphoreType.DMA,
    ],
)
def _sc_gather(ht_hbm, ctr_hbm, ctx_hbm, out_hbm,
               ctr_v, ctx_v, hrows_v, acc_v, sem):
    wid = lax.axis_index("s") * _NC + lax.axis_index("c")
    base = wid * BW
    pltpu.sync_copy(ctr_hbm.at[pl.ds(base, BW)], ctr_v)
    cp = pltpu.async_copy(ht_hbm.at[ctr_v], hrows_v, sem)
    pltpu.sync_copy(ctx_hbm.at[pl.ds(base, BW)], ctx_v)
    cp.wait()

    def body(b, acc):
        bvec = jnp.full((LN,), b, jnp.int32)
        i0 = ctx_v[b, pl.ds(0, LN)]
        i1 = ctx_v[b, pl.ds(LN, LN)]
        v0 = plsc.load_gather(hrows_v, [bvec, i0])
        v1 = plsc.load_gather(hrows_v, [bvec, i1])
        return acc + v0 + v1

    acc_v[...] = lax.fori_loop(0, BW, body, jnp.zeros((LN,), jnp.float32))
    pltpu.sync_copy(acc_v, out_hbm.at[wid])


def _fin_body(p_ref, o_ref):
    o_ref[...] = jnp.sum(p_ref[...]).reshape(1, 1) * (1.0 / N)


_fin_call = pl.pallas_call(
    _fin_body,
    out_shape=jax.ShapeDtypeStruct((1, 1), jnp.float32),
)


def kernel(center, context, emb_in, W_out):
    emb_p = jnp.zeros((VP, D), jnp.float32).at[:V].set(emb_in)
    w_p = jnp.zeros((VP, D), jnp.float32).at[:V].set(W_out)
    ctx_p = jnp.pad(context.astype(jnp.int32), ((0, 0), (0, LP - L)),
                    constant_values=V)
    ht = _ht_call(emb_p, w_p)
    parts = _sc_gather(ht, center.astype(jnp.int32), ctx_p)
    return _fin_call(parts)[0, 0]


# speedup vs baseline: 4.6157x; 1.1673x over previous
import functools

import jax
import jax.numpy as jnp
from jax import lax
from jax.experimental import pallas as pl
from jax.experimental.pallas import tpu as pltpu
from jax.experimental.pallas import tpu_sc as plsc

V = 1000
D = 128
B = 1024
L = 20
VP = 1024
N = B * L

_INFO = plsc.get_sparse_core_info()
_NC = _INFO.num_cores
_NS = _INFO.num_subcores
LN = _INFO.num_lanes
NW = _NC * _NS
BW = B // NW


def _ht_body(emb_ref, w_ref, ht_ref):
    zpad = jnp.zeros((VP - V, D), jnp.float32)
    wp = jnp.concatenate([w_ref[...], zpad], axis=0)
    ep = jnp.concatenate([emb_ref[...], zpad], axis=0)
    a = lax.dot_general(wp, ep, (((1,), (1,)), ((), ())),
                        preferred_element_type=jnp.float32)
    row_c = lax.broadcasted_iota(jnp.int32, (VP, VP), 0)
    a_msk = jnp.where(row_c < V, a, -1e30)
    m = jnp.max(a_msk, axis=0, keepdims=True)
    lz = m + jnp.log(jnp.sum(jnp.exp(a_msk - m), axis=0, keepdims=True))
    ht_ref[...] = lz - a


_ht_call = pl.pallas_call(
    _ht_body,
    out_shape=jax.ShapeDtypeStruct((VP, VP), jnp.float32),
)


_mesh = plsc.VectorSubcoreMesh(core_axis_name="c", subcore_axis_name="s")


@functools.partial(
    pl.kernel,
    mesh=_mesh,
    compiler_params=pltpu.CompilerParams(use_tc_tiling_on_sc=False,
                                         needs_layout_passes=False),
    out_type=jax.ShapeDtypeStruct((NW * LN,), jnp.float32),
    scratch_types=[
        pltpu.VMEM((BW,), jnp.int32),
        pltpu.VMEM((BW, L), jnp.int32),
        pltpu.VMEM((BW, VP), jnp.float32),
        pltpu.VMEM((LN,), jnp.float32),
        pltpu.SemaphoreType.DMA,
    ],
)
def _sc_gather(ht_hbm, ctr_hbm, ctx_hbm, out_hbm,
               ctr_v, ctx_v, hrows_v, acc_v, sem):
    wid = lax.axis_index("s") * _NC + lax.axis_index("c")
    base = wid * BW
    pltpu.sync_copy(ctr_hbm.at[pl.ds(base, BW)], ctr_v)
    cp = pltpu.async_copy(ht_hbm.at[ctr_v], hrows_v, sem)
    pltpu.sync_copy(ctx_hbm.at[pl.ds(base, BW)], ctx_v)
    cp.wait()

    lane = lax.iota(jnp.int32, LN)
    tail = (lane >= LN - (L - LN)).astype(jnp.float32)

    def body(b, acc):
        bvec = jnp.full((LN,), b, jnp.int32)
        i0 = ctx_v[b, pl.ds(0, LN)]
        i1 = ctx_v[b, pl.ds(L - LN, LN)]
        v0 = plsc.load_gather(hrows_v, [bvec, i0])
        v1 = plsc.load_gather(hrows_v, [bvec, i1])
        return acc + v0 + v1 * tail

    acc_v[...] = lax.fori_loop(0, BW, body, jnp.zeros((LN,), jnp.float32))
    pltpu.sync_copy(acc_v, out_hbm.at[pl.ds(wid * LN, LN)])


def _fin_body(p_ref, o_ref):
    o_ref[...] = jnp.sum(p_ref[...]).reshape(1, 1) * (1.0 / N)


_fin_call = pl.pallas_call(
    _fin_body,
    out_shape=jax.ShapeDtypeStruct((1, 1), jnp.float32),
)


def kernel(center, context, emb_in, W_out):
    ht = _ht_call(emb_in, W_out)
    parts = _sc_gather(ht, center.astype(jnp.int32), context.astype(jnp.int32))
    return _fin_call(parts)[0, 0]


# speedup vs baseline: 5.3507x; 1.1592x over previous
import functools

import jax
import jax.numpy as jnp
from jax import lax
from jax.experimental import pallas as pl
from jax.experimental.pallas import tpu as pltpu
from jax.experimental.pallas import tpu_sc as plsc

V = 1000
D = 128
B = 1024
L = 20
VP = 1024
N = B * L

_INFO = plsc.get_sparse_core_info()
_NC = _INFO.num_cores
_NS = _INFO.num_subcores
LN = _INFO.num_lanes
NW = _NC * _NS
BW = B // NW


def _ht_body(emb_ref, w_ref, ht_ref):
    zpad = jnp.zeros((VP - V, D), jnp.float32)
    wp = jnp.concatenate([w_ref[...], zpad], axis=0)
    ep = jnp.concatenate([emb_ref[...], zpad], axis=0)
    a = lax.dot_general(wp, ep, (((1,), (1,)), ((), ())),
                        preferred_element_type=jnp.float32)
    row_c = lax.broadcasted_iota(jnp.int32, (VP, VP), 0)
    a_msk = jnp.where(row_c < V, a, -1e30)
    m = jnp.max(a_msk, axis=0, keepdims=True)
    lz = m + jnp.log(jnp.sum(jnp.exp(a_msk - m), axis=0, keepdims=True))
    h = lz - a
    for k in range(VP // 128):
        ht_ref[k, :, :] = h[:, 128 * k:128 * (k + 1)]


_ht_call = pl.pallas_call(
    _ht_body,
    out_shape=jax.ShapeDtypeStruct((VP // 128, VP, 128), jnp.float32),
)


_mesh = plsc.VectorSubcoreMesh(core_axis_name="c", subcore_axis_name="s")


@functools.partial(
    pl.kernel,
    mesh=_mesh,
    compiler_params=pltpu.CompilerParams(use_tc_tiling_on_sc=False,
                                         needs_layout_passes=False),
    out_type=jax.ShapeDtypeStruct((NW * LN,), jnp.float32),
    scratch_types=[
        pltpu.VMEM((BW,), jnp.int32),
        pltpu.VMEM((BW, L), jnp.int32),
        pltpu.VMEM((VP // 128, BW, 128), jnp.float32),
        pltpu.VMEM((LN,), jnp.float32),
        pltpu.SemaphoreType.DMA,
    ],
)
def _sc_gather(ht_hbm, ctr_hbm, ctx_hbm, out_hbm,
               ctr_v, ctx_v, hrows_v, acc_v, sem):
    wid = lax.axis_index("s") * _NC + lax.axis_index("c")
    base = wid * BW
    pltpu.sync_copy(ctr_hbm.at[pl.ds(base, BW)], ctr_v)
    cps = [pltpu.async_copy(ht_hbm.at[k].at[ctr_v], hrows_v.at[k], sem)
           for k in range(VP // 128)]
    pltpu.sync_copy(ctx_hbm.at[pl.ds(base, BW)], ctx_v)
    for cp in cps:
        cp.wait()

    lane = lax.iota(jnp.int32, LN)
    tail = (lane >= LN - (L - LN)).astype(jnp.float32)

    def body(b, acc):
        bvec = jnp.full((LN,), b, jnp.int32)
        i0 = ctx_v[b, pl.ds(0, LN)]
        i1 = ctx_v[b, pl.ds(L - LN, LN)]
        v0 = plsc.load_gather(hrows_v, [i0 >> 7, bvec, i0 & 127])
        v1 = plsc.load_gather(hrows_v, [i1 >> 7, bvec, i1 & 127])
        return acc + v0 + v1 * tail

    acc_v[...] = lax.fori_loop(0, BW, body, jnp.zeros((LN,), jnp.float32))
    pltpu.sync_copy(acc_v, out_hbm.at[pl.ds(wid * LN, LN)])


def _fin_body(p_ref, o_ref):
    o_ref[...] = jnp.sum(p_ref[...]).reshape(1, 1) * (1.0 / N)


_fin_call = pl.pallas_call(
    _fin_body,
    out_shape=jax.ShapeDtypeStruct((1, 1), jnp.float32),
)


def kernel(center, context, emb_in, W_out):
    ht = _ht_call(emb_in, W_out)
    parts = _sc_gather(ht, center.astype(jnp.int32), context.astype(jnp.int32))
    return _fin_call(parts)[0, 0]


# speedup vs baseline: 5.4376x; 1.0162x over previous
import functools

import jax
import jax.numpy as jnp
from jax import lax
from jax.experimental import pallas as pl
from jax.experimental.pallas import tpu as pltpu
from jax.experimental.pallas import tpu_sc as plsc

V = 1000
D = 128
B = 1024
L = 20
VP = 1024
N = B * L

_INFO = plsc.get_sparse_core_info()
_NC = _INFO.num_cores
_NS = _INFO.num_subcores
LN = _INFO.num_lanes
NW = _NC * _NS
BW = B // NW


def _ht_body(emb_ref, w_ref, ht_ref):
    zpad = jnp.zeros((VP - V, D), jnp.float32)
    wp = jnp.concatenate([w_ref[...], zpad], axis=0)
    ep = jnp.concatenate([emb_ref[...], zpad], axis=0)
    a = lax.dot_general(wp, ep, (((1,), (1,)), ((), ())),
                        preferred_element_type=jnp.float32)
    row_c = lax.broadcasted_iota(jnp.int32, (VP, VP), 0)
    a_msk = jnp.where(row_c < V, a, -1e30)
    m = jnp.max(a_msk, axis=0, keepdims=True)
    lz = m + jnp.log(jnp.sum(jnp.exp(a_msk - m), axis=0, keepdims=True))
    h = lz - a
    for k in range(VP // 128):
        ht_ref[k, :, :] = h[:, 128 * k:128 * (k + 1)]


_ht_call = pl.pallas_call(
    _ht_body,
    out_shape=jax.ShapeDtypeStruct((VP // 128, VP, 128), jnp.float32),
)


_mesh = plsc.VectorSubcoreMesh(core_axis_name="c", subcore_axis_name="s")


@functools.partial(
    pl.kernel,
    mesh=_mesh,
    compiler_params=pltpu.CompilerParams(use_tc_tiling_on_sc=False,
                                         needs_layout_passes=False),
    out_type=jax.ShapeDtypeStruct((NW * LN,), jnp.float32),
    scratch_types=[
        pltpu.VMEM((BW,), jnp.int32),
        pltpu.VMEM((BW, L), jnp.int32),
        pltpu.VMEM((VP // 128, BW, 128), jnp.float32),
        pltpu.VMEM((LN,), jnp.float32),
        pltpu.SemaphoreType.DMA,
    ],
)
def _sc_gather(ht_hbm, ctr_hbm, ctx_hbm, out_hbm,
               ctr_v, ctx_v, hrows_v, acc_v, sem):
    wid = lax.axis_index("s") * _NC + lax.axis_index("c")
    base = wid * BW
    pltpu.sync_copy(ctr_hbm.at[pl.ds(base, BW)], ctr_v)
    cps = [pltpu.async_copy(ht_hbm.at[k].at[ctr_v], hrows_v.at[k], sem)
           for k in range(VP // 128)]
    pltpu.sync_copy(ctx_hbm.at[pl.ds(base, BW)], ctx_v)
    for cp in cps:
        cp.wait()

    lane = lax.iota(jnp.int32, LN)
    tail = (lane >= LN - (L - LN)).astype(jnp.float32)

    def body(b, acc):
        bvec = jnp.full((LN,), b, jnp.int32)
        i0 = ctx_v[b, pl.ds(0, LN)]
        i1 = ctx_v[b, pl.ds(L - LN, LN)]
        v0 = plsc.load_gather(hrows_v, [i0 >> 7, bvec, i0 & 127])
        v1 = plsc.load_gather(hrows_v, [i1 >> 7, bvec, i1 & 127])
        return acc + v0 + v1 * tail

    acc_v[...] = lax.fori_loop(0, BW, body, jnp.zeros((LN,), jnp.float32))
    pltpu.sync_copy(acc_v, out_hbm.at[pl.ds(wid * LN, LN)])


def _fin_body(p_ref, o_ref):
    o_ref[...] = jnp.sum(p_ref[...]).reshape(1, 1) * (1.0 / N)


_fin_call = pl.pallas_call(
    _fin_body,
    out_shape=jax.ShapeDtypeStruct((1, 1), jnp.float32),
)


def kernel(center, context, emb_in, W_out):
    ht = _ht_call(emb_in, W_out)
    parts = _sc_gather(ht, center, context)
    return _fin_call(parts)[0, 0]


# speedup vs baseline: 5.4624x; 1.0046x over previous
import functools

import jax
import jax.numpy as jnp
from jax import lax
from jax.experimental import pallas as pl
from jax.experimental.pallas import tpu as pltpu
from jax.experimental.pallas import tpu_sc as plsc

V = 1000
D = 128
B = 1024
L = 20
VP = 1024
KC = VP // 128
N = B * L

_INFO = plsc.get_sparse_core_info()
_NC = _INFO.num_cores
_NS = _INFO.num_subcores
LN = _INFO.num_lanes
NW = _NC * _NS
RW = N // 128 // NW


def _ht_body(emb_ref, w_ref, ctr_ref, ctx_ref, ht_ref, fidx_ref):
    zpad = jnp.zeros((VP - V, D), jnp.float32)
    wp = jnp.concatenate([w_ref[...], zpad], axis=0)
    ep = jnp.concatenate([emb_ref[...], zpad], axis=0)
    a = lax.dot_general(wp, ep, (((1,), (1,)), ((), ())),
                        preferred_element_type=jnp.float32)
    row_c = lax.broadcasted_iota(jnp.int32, (VP, VP), 0)
    a_msk = jnp.where(row_c < V, a, -1e30)
    m = jnp.max(a_msk, axis=0, keepdims=True)
    lz = m + jnp.log(jnp.sum(jnp.exp(a_msk - m), axis=0, keepdims=True))
    h = lz - a
    for k in range(KC):
        ht_ref[k, :, :] = h[:, 128 * k:128 * (k + 1)]
    ctx = ctx_ref[...]
    ctr = ctr_ref[...].reshape(B, 1)
    f = (ctx >> 7) * (VP * 128) + ctr * 128 + (ctx & 127)
    fidx_ref[...] = jnp.swapaxes(f, 0, 1).reshape(N // 128, 128)


_ht_call = pl.pallas_call(
    _ht_body,
    out_shape=(jax.ShapeDtypeStruct((KC, VP, 128), jnp.float32),
               jax.ShapeDtypeStruct((N // 128, 128), jnp.int32)),
)


_mesh = plsc.VectorSubcoreMesh(core_axis_name="c", subcore_axis_name="s")


@functools.partial(
    pl.kernel,
    mesh=_mesh,
    compiler_params=pltpu.CompilerParams(use_tc_tiling_on_sc=False,
                                         needs_layout_passes=False),
    out_type=jax.ShapeDtypeStruct((NW * LN,), jnp.float32),
    scratch_types=[
        pltpu.VMEM((RW, 128), jnp.int32),
        pltpu.VMEM((RW, 128), jnp.float32),
        pltpu.VMEM((LN,), jnp.float32),
        pltpu.SemaphoreType.DMA,
    ],
)
def _sc_gather(htf_hbm, fidx_hbm, out_hbm, idx_v, hv, acc_v, sem):
    wid = lax.axis_index("s") * _NC + lax.axis_index("c")
    base = wid * RW
    pltpu.sync_copy(fidx_hbm.at[pl.ds(base, RW)], idx_v)
    cps = [pltpu.async_copy(htf_hbm.at[idx_v.at[r]], hv.at[r], sem)
           for r in range(RW)]
    for cp in cps:
        cp.wait()
    acc = jnp.zeros((LN,), jnp.float32)
    for r in range(RW):
        for c in range(128 // LN):
            acc = acc + hv[r, pl.ds(c * LN, LN)]
    acc_v[...] = acc
    pltpu.sync_copy(acc_v, out_hbm.at[pl.ds(wid * LN, LN)])


def _fin_body(p_ref, o_ref):
    o_ref[...] = jnp.sum(p_ref[...]).reshape(1, 1) * (1.0 / N)


_fin_call = pl.pallas_call(
    _fin_body,
    out_shape=jax.ShapeDtypeStruct((1, 1), jnp.float32),
)


def kernel(center, context, emb_in, W_out):
    ht, fidx = _ht_call(emb_in, W_out, center, context)
    parts = _sc_gather(ht.reshape(KC * VP * 128), fidx)
    return _fin_call(parts)[0, 0]
